# Initial kernel scaffold; baseline (speedup 1.0000x reference)
import functools

import jax
import jax.numpy as jnp
from jax import lax
from jax.experimental import pallas as pl
from jax.experimental.pallas import tpu as pltpu
from jax.experimental.pallas import tpu_sc as plsc

_DIM = 128
_NC = 2
_NS = 16
_NW = _NC * _NS
_CHUNK = 128


def _sc_gather_fn(n_ent, n_rel):
    e_rows_w = n_ent // _NW
    r_rows_w = n_rel // _NW
    e_chunks = e_rows_w // _CHUNK
    r_chunks = r_rows_w // _CHUNK
    mesh = plsc.VectorSubcoreMesh(core_axis_name="c", subcore_axis_name="s")

    @functools.partial(
        pl.kernel,
        out_type=[
            jax.ShapeDtypeStruct((n_ent, _DIM), jnp.float32),
            jax.ShapeDtypeStruct((n_rel, _DIM), jnp.float32),
        ],
        mesh=mesh,
        scratch_types=[
            pltpu.VMEM((e_chunks, _CHUNK), jnp.int32),
            pltpu.VMEM((r_chunks, _CHUNK), jnp.int32),
            pltpu.VMEM((e_rows_w, _DIM), jnp.float32),
            pltpu.VMEM((r_rows_w, _DIM), jnp.float32),
            pltpu.SemaphoreType.DMA,
        ],
    )
    def gather(ent_hbm, rel_hbm, ie_hbm, ir_hbm, oe_hbm, or_hbm,
               ie_v, ir_v, erows_v, rrows_v, sem):
        wid = lax.axis_index("s") * _NC + lax.axis_index("c")
        pltpu.sync_copy(ie_hbm.at[pl.ds(wid * e_chunks, e_chunks)], ie_v)
        pltpu.sync_copy(ir_hbm.at[pl.ds(wid * r_chunks, r_chunks)], ir_v)
        copies = []
        for j in range(e_chunks):
            copies.append(pltpu.async_copy(
                ent_hbm.at[ie_v.at[j]],
                erows_v.at[pl.ds(j * _CHUNK, _CHUNK)], sem))
        for j in range(r_chunks):
            copies.append(pltpu.async_copy(
                rel_hbm.at[ir_v.at[j]],
                rrows_v.at[pl.ds(j * _CHUNK, _CHUNK)], sem))
        for c in copies:
            c.wait()
        pltpu.sync_copy(erows_v, oe_hbm.at[pl.ds(wid * e_rows_w, e_rows_w)])
        pltpu.sync_copy(rrows_v, or_hbm.at[pl.ds(wid * r_rows_w, r_rows_w)])

    return gather


def _tc_loss(erows_ref, rrows_ref, out_ref):
    b = rrows_ref.shape[0] // 2

    def unit(x):
        n = jnp.sqrt(jnp.sum(x * x, axis=1, keepdims=True))
        return x / jnp.maximum(n, 1e-12)

    hp = unit(erows_ref[0:b, :])
    tp = unit(erows_ref[b:2 * b, :])
    hn = unit(erows_ref[2 * b:3 * b, :])
    tn = unit(erows_ref[3 * b:4 * b, :])
    rp = unit(rrows_ref[0:b, :])
    rn = unit(rrows_ref[b:2 * b, :])

    dp = hp + rp - tp
    dn = hn + rn - tn
    ep = jnp.sqrt(jnp.sum(dp * dp, axis=1))
    en = jnp.sqrt(jnp.sum(dn * dn, axis=1))
    loss = jnp.maximum(1.0 + ep - en, 0.0)
    out_ref[0, 0] = jnp.sum(loss) * (1.0 / b)


@jax.jit
def kernel(pos_triples, neg_triples, ent_emb, rel_emb):
    b = pos_triples.shape[0]
    idx_ent = jnp.concatenate([
        pos_triples[:, 0], pos_triples[:, 2],
        neg_triples[:, 0], neg_triples[:, 2],
    ]).reshape(-1, _CHUNK)
    idx_rel = jnp.concatenate([
        pos_triples[:, 1], neg_triples[:, 1],
    ]).reshape(-1, _CHUNK)

    erows, rrows = _sc_gather_fn(4 * b, 2 * b)(
        ent_emb, rel_emb, idx_ent, idx_rel)

    out = pl.pallas_call(
        _tc_loss,
        out_shape=jax.ShapeDtypeStruct((1, 1), jnp.float32),
    )(erows, rrows)
    return out[0, 0]


# speedup vs baseline: 1.8155x; 1.8155x over previous
import functools

import jax
import jax.numpy as jnp
from jax import lax
from jax.experimental import pallas as pl
from jax.experimental.pallas import tpu as pltpu
from jax.experimental.pallas import tpu_sc as plsc

_DIM = 128
_NC = 2
_NS = 16
_NW = _NC * _NS
_CHUNK = 128


def _sc_gather_fn(n_ent, n_rel):
    e_rows_w = n_ent // _NW
    r_rows_w = n_rel // _NW
    e_chunks = e_rows_w // _CHUNK
    r_chunks = r_rows_w // _CHUNK
    mesh = plsc.VectorSubcoreMesh(core_axis_name="c", subcore_axis_name="s")

    @functools.partial(
        pl.kernel,
        out_type=[
            jax.ShapeDtypeStruct((n_ent, _DIM), jnp.float32),
            jax.ShapeDtypeStruct((n_rel, _DIM), jnp.float32),
        ],
        mesh=mesh,
        scratch_types=[
            pltpu.VMEM((e_chunks, _CHUNK), jnp.int32),
            pltpu.VMEM((r_chunks, _CHUNK), jnp.int32),
            pltpu.VMEM((e_rows_w, _DIM), jnp.float32),
            pltpu.VMEM((r_rows_w, _DIM), jnp.float32),
            pltpu.SemaphoreType.DMA,
        ],
    )
    def gather(ent_hbm, rel_hbm, ie_hbm, ir_hbm, oe_hbm, or_hbm,
               ie_v, ir_v, erows_v, rrows_v, sem):
        wid = lax.axis_index("s") * _NC + lax.axis_index("c")
        pltpu.sync_copy(ie_hbm.at[pl.ds(wid * e_chunks, e_chunks)], ie_v)
        pltpu.sync_copy(ir_hbm.at[pl.ds(wid * r_chunks, r_chunks)], ir_v)
        copies = []
        for j in range(e_chunks):
            copies.append(pltpu.async_copy(
                ent_hbm.at[ie_v.at[j]],
                erows_v.at[pl.ds(j * _CHUNK, _CHUNK)], sem))
        for j in range(r_chunks):
            copies.append(pltpu.async_copy(
                rel_hbm.at[ir_v.at[j]],
                rrows_v.at[pl.ds(j * _CHUNK, _CHUNK)], sem))
        for c in copies:
            c.wait()
        pltpu.sync_copy(erows_v, oe_hbm.at[pl.ds(wid * e_rows_w, e_rows_w)])
        pltpu.sync_copy(rrows_v, or_hbm.at[pl.ds(wid * r_rows_w, r_rows_w)])

    return gather


def _tc_loss(erows_ref, rrows_ref, out_ref):
    b = rrows_ref.shape[0] // 2

    def unit(x):
        n = jnp.sqrt(jnp.sum(x * x, axis=1, keepdims=True))
        return x / jnp.maximum(n, 1e-12)

    hp = unit(erows_ref[0:b, :])
    tp = unit(erows_ref[b:2 * b, :])
    hn = unit(erows_ref[2 * b:3 * b, :])
    tn = unit(erows_ref[3 * b:4 * b, :])
    rp = unit(rrows_ref[0:b, :])
    rn = unit(rrows_ref[b:2 * b, :])

    dp = hp + rp - tp
    dn = hn + rn - tn
    ep = jnp.sqrt(jnp.sum(dp * dp, axis=1))
    en = jnp.sqrt(jnp.sum(dn * dn, axis=1))
    loss = jnp.maximum(1.0 + ep - en, 0.0)
    out_ref[...] = (jnp.sum(loss) * (1.0 / b)).reshape(1, 1)


@jax.jit
def kernel(pos_triples, neg_triples, ent_emb, rel_emb):
    b = pos_triples.shape[0]
    idx_ent = jnp.concatenate([
        pos_triples[:, 0], pos_triples[:, 2],
        neg_triples[:, 0], neg_triples[:, 2],
    ]).reshape(-1, _CHUNK)
    idx_rel = jnp.concatenate([
        pos_triples[:, 1], neg_triples[:, 1],
    ]).reshape(-1, _CHUNK)

    erows, rrows = _sc_gather_fn(4 * b, 2 * b)(
        ent_emb, rel_emb, idx_ent, idx_rel)

    out = pl.pallas_call(
        _tc_loss,
        out_shape=jax.ShapeDtypeStruct((1, 1), jnp.float32),
    )(erows, rrows)
    return out[0, 0]
